# baseline (speedup 1.0000x reference)
import functools

import jax
import jax.numpy as jnp
from jax import lax
from jax.experimental import pallas as pl
from jax.experimental.pallas import tpu as pltpu
from jax.experimental.pallas import tpu_sc as plsc

N_NODES = 100000
N_EDGES = 6400000
NP = 100352
ROWS = NP // 128
C = 128
K = 10
NC = 2
NS = 16
NW = NC * NS
TOTAL_CHUNKS = N_EDGES // C
TOTAL_GROUPS = TOTAL_CHUNKS // K
BASE_G = TOTAL_GROUPS // NW
EXTRA_G = TOTAL_GROUPS - BASE_G * NW
SLICE = NP // NS
GE = K * C


def _edge_accumulate(ei, d_pad, zeros2):
    mesh = plsc.VectorSubcoreMesh(core_axis_name="c", subcore_axis_name="s")

    @functools.partial(
        pl.kernel,
        out_type=jax.ShapeDtypeStruct((NC, 2 * NP), jnp.float32),
        mesh=mesh,
        compiler_params=pltpu.CompilerParams(needs_layout_passes=False),
        scratch_types=[
            pltpu.VMEM((NP,), jnp.float32),
            pltpu.VMEM((GE,), jnp.int32),
            pltpu.VMEM((GE,), jnp.int32),
            pltpu.VMEM((2 * GE,), jnp.int32),
            pltpu.VMEM((2 * GE,), jnp.float32),
            pltpu.VMEM((GE,), jnp.int32),
            pltpu.VMEM((GE,), jnp.int32),
            pltpu.VMEM((2 * GE,), jnp.int32),
            pltpu.VMEM((2 * GE,), jnp.float32),
            pltpu.VMEM_SHARED((2 * NP,), jnp.float32),
            pltpu.SemaphoreType.DMA,
            pltpu.SemaphoreType.DMA,
        ],
    )
    def k(ei_hbm, d_hbm, z_hbm, ad_out,
          d_v, src_a, dst_a, ib_a, vb_a, src_b, dst_b, ib_b, vb_b,
          ad_sh, ssem, lsem):
        cid = lax.axis_index("c")
        sid = lax.axis_index("s")
        wid = sid * NC + cid

        off2 = sid * (2 * SLICE)
        pltpu.sync_copy(z_hbm.at[pl.ds(off2, 2 * SLICE)],
                        ad_sh.at[pl.ds(off2, 2 * SLICE)])
        pltpu.sync_copy(d_hbm, d_v)
        iota2 = lax.iota(jnp.int32, 16) * 2
        one_i = jnp.ones((16,), jnp.int32)
        fones = jnp.ones((16,), jnp.float32)
        for t in range(GE // 16):
            podd = iota2 + (t * 32 + 1)
            plsc.store_scatter(vb_a, [podd], fones)
            plsc.store_scatter(vb_b, [podd], fones)
        plsc.subcore_barrier()

        g0 = wid * BASE_G + jnp.minimum(wid, EXTRA_G)
        n_groups = BASE_G + (wid < EXTRA_G).astype(jnp.int32)

        def loadgather(src_r, dst_r, ib_r, vb_r, g_rel):
            base = (g0 + g_rel) * GE
            h1 = pltpu.async_copy(ei_hbm.at[0, pl.ds(base, GE)], src_r, lsem)
            h2 = pltpu.async_copy(ei_hbm.at[1, pl.ds(base, GE)], dst_r, lsem)
            h1.wait()
            h2.wait()
            for t in range(GE // 16):
                pe = iota2 + (t * 32)
                dstv = dst_r[pl.ds(t * 16, 16)]
                e = dstv + dstv
                plsc.store_scatter(ib_r, [pe], e)
                plsc.store_scatter(ib_r, [pe + one_i], e + one_i)
                srcv = src_r[pl.ds(t * 16, 16)]
                v = plsc.load_gather(d_v, [srcv])
                plsc.store_scatter(vb_r, [pe], v)

        def fire(ib_r, vb_r):
            for j in range(2 * K):
                pltpu.async_copy(vb_r.at[pl.ds(j * C, C)],
                                 ad_sh.at[ib_r.at[pl.ds(j * C, C)]],
                                 ssem, add=True)

        def drain():
            for _ in range(2 * K):
                pltpu.make_async_copy(d_hbm.at[pl.ds(0, C)],
                                      vb_a.at[pl.ds(0, C)], ssem).wait()

        n_pairs = n_groups // 2
        tail = n_groups - 2 * n_pairs

        @pl.loop(0, n_pairs)
        def _(gg):
            loadgather(src_a, dst_a, ib_a, vb_a, 2 * gg)

            @pl.when(gg > 0)
            def _():
                drain()

            fire(ib_a, vb_a)
            loadgather(src_b, dst_b, ib_b, vb_b, 2 * gg + 1)
            drain()
            fire(ib_b, vb_b)

        drain()

        @pl.when(tail == 1)
        def _():
            loadgather(src_a, dst_a, ib_a, vb_a, 2 * n_pairs)
            fire(ib_a, vb_a)
            drain()

        plsc.subcore_barrier()
        pltpu.sync_copy(ad_sh.at[pl.ds(off2, 2 * SLICE)],
                        ad_out.at[cid, pl.ds(off2, 2 * SLICE)])

    return k(ei, d_pad, zeros2)


def _loss_body(acc_ref, deg_ref, d_ref, out_ref):
    acc = acc_ref[0] + acc_ref[1]
    deg = deg_ref[0] + deg_ref[1]
    d = d_ref[...]
    w = (deg > 0).astype(jnp.float32)
    vort = acc / jnp.maximum(deg, 1.0) - d
    num = jnp.sum(jnp.abs(vort) * w)
    den = jnp.maximum(jnp.sum(w), 1.0)
    out_ref[...] = jnp.full((1, 1), num / den, jnp.float32)


def _masked_loss(acc_p, deg_p, d_pad):
    out = pl.pallas_call(
        _loss_body,
        out_shape=jax.ShapeDtypeStruct((1, 1), jnp.float32),
    )(acc_p.reshape(NC, ROWS, 128), deg_p.reshape(NC, ROWS, 128),
      d_pad.reshape(ROWS, 128))
    return out[0, 0]


@jax.jit
def kernel(u_out, v_out, u_tgt, v_tgt, edge_index):
    d = (v_out - v_tgt) - (u_out - u_tgt)
    d_pad = jnp.zeros((NP,), jnp.float32).at[:N_NODES].set(d)
    ei = edge_index.astype(jnp.int32)
    zeros2 = jnp.zeros((2 * NP,), jnp.float32)
    ad = _edge_accumulate(ei, d_pad, zeros2)
    pairs = ad.reshape(NC, NP, 2)
    return _masked_loss(pairs[..., 0], pairs[..., 1], d_pad)


# speedup vs baseline: 2.4540x; 2.4540x over previous
import functools

import jax
import jax.numpy as jnp
from jax import lax
from jax.experimental import pallas as pl
from jax.experimental.pallas import tpu as pltpu
from jax.experimental.pallas import tpu_sc as plsc

N_NODES = 100000
N_EDGES = 6400000
NP = 100352
ROWS = NP // 128
C = 128
K = 20
NC = 2
NS = 16
NW = NC * NS
TOTAL_CHUNKS = N_EDGES // C
TOTAL_GROUPS = TOTAL_CHUNKS // K
BASE_G = TOTAL_GROUPS // NW
EXTRA_G = TOTAL_GROUPS - BASE_G * NW
SLICE = NP // NS


def _edge_accumulate(ei, d_pad, zeros):
    mesh = plsc.VectorSubcoreMesh(core_axis_name="c", subcore_axis_name="s")

    @functools.partial(
        pl.kernel,
        out_type=(
            jax.ShapeDtypeStruct((NC, NP), jnp.float32),
            jax.ShapeDtypeStruct((NC, NP), jnp.float32),
        ),
        mesh=mesh,
        compiler_params=pltpu.CompilerParams(needs_layout_passes=False),
        scratch_types=[
            pltpu.VMEM((NP,), jnp.float32),
            pltpu.VMEM((K * C,), jnp.int32),
            pltpu.VMEM((K * C,), jnp.int32),
            pltpu.VMEM((K * C,), jnp.float32),
            pltpu.VMEM((K * C,), jnp.int32),
            pltpu.VMEM((K * C,), jnp.int32),
            pltpu.VMEM((K * C,), jnp.float32),
            pltpu.VMEM((C,), jnp.float32),
            pltpu.VMEM_SHARED((NP,), jnp.float32),
            pltpu.VMEM_SHARED((NP,), jnp.float32),
            pltpu.SemaphoreType.DMA,
            pltpu.SemaphoreType.DMA,
        ],
    )
    def k(ei_hbm, d_hbm, z_hbm, acc_out, deg_out,
          d_v, src_a, dst_a, vals_a, src_b, dst_b, vals_b,
          ones, acc_sh, deg_sh, ssem, lsem):
        cid = lax.axis_index("c")
        sid = lax.axis_index("s")
        wid = sid * NC + cid

        off = sid * SLICE
        h1 = pltpu.async_copy(z_hbm.at[pl.ds(off, SLICE)],
                              acc_sh.at[pl.ds(off, SLICE)], lsem)
        h2 = pltpu.async_copy(z_hbm.at[pl.ds(off, SLICE)],
                              deg_sh.at[pl.ds(off, SLICE)], lsem)
        h3 = pltpu.async_copy(d_hbm, d_v, lsem)
        for j in range(C // 16):
            ones[pl.ds(j * 16, 16)] = jnp.ones((16,), jnp.float32)
        h1.wait()
        h2.wait()
        h3.wait()
        plsc.subcore_barrier()

        g0 = wid * BASE_G + jnp.minimum(wid, EXTRA_G)
        n_groups = BASE_G + (wid < EXTRA_G).astype(jnp.int32)

        def loadgather(src_r, dst_r, vals_r, g_rel):
            base = (g0 + g_rel) * (K * C)
            h1 = pltpu.async_copy(ei_hbm.at[0, pl.ds(base, K * C)], src_r, lsem)
            h2 = pltpu.async_copy(ei_hbm.at[1, pl.ds(base, K * C)], dst_r, lsem)
            h1.wait()
            h2.wait()
            for t in range(K * C // 16):
                idx = src_r[pl.ds(t * 16, 16)]
                vals_r[pl.ds(t * 16, 16)] = plsc.load_gather(d_v, [idx])

        def fire(dst_r, vals_r):
            for j in range(K):
                pltpu.async_copy(vals_r.at[pl.ds(j * C, C)],
                                 acc_sh.at[dst_r.at[pl.ds(j * C, C)]],
                                 ssem, add=True)
                pltpu.async_copy(ones, deg_sh.at[dst_r.at[pl.ds(j * C, C)]],
                                 ssem, add=True)

        def drain():
            for _ in range(2 * K):
                pltpu.make_async_copy(d_hbm.at[pl.ds(0, C)], ones, ssem).wait()

        n_pairs = n_groups // 2
        tail = n_groups - 2 * n_pairs

        @pl.loop(0, n_pairs)
        def _(gg):
            loadgather(src_a, dst_a, vals_a, 2 * gg)

            @pl.when(gg > 0)
            def _():
                drain()

            fire(dst_a, vals_a)
            loadgather(src_b, dst_b, vals_b, 2 * gg + 1)
            drain()
            fire(dst_b, vals_b)

        drain()

        @pl.when(tail == 1)
        def _():
            loadgather(src_a, dst_a, vals_a, 2 * n_pairs)
            fire(dst_a, vals_a)
            drain()

        plsc.subcore_barrier()
        h1 = pltpu.async_copy(acc_sh.at[pl.ds(off, SLICE)],
                              acc_out.at[cid, pl.ds(off, SLICE)], lsem)
        h2 = pltpu.async_copy(deg_sh.at[pl.ds(off, SLICE)],
                              deg_out.at[cid, pl.ds(off, SLICE)], lsem)
        h1.wait()
        h2.wait()

    return k(ei, d_pad, zeros)


def _loss_body(acc_ref, deg_ref, d_ref, out_ref):
    acc = acc_ref[0] + acc_ref[1]
    deg = deg_ref[0] + deg_ref[1]
    d = d_ref[...]
    w = (deg > 0).astype(jnp.float32)
    vort = acc / jnp.maximum(deg, 1.0) - d
    num = jnp.sum(jnp.abs(vort) * w)
    den = jnp.maximum(jnp.sum(w), 1.0)
    out_ref[...] = jnp.full((1, 1), num / den, jnp.float32)


def _masked_loss(acc_p, deg_p, d_pad):
    out = pl.pallas_call(
        _loss_body,
        out_shape=jax.ShapeDtypeStruct((1, 1), jnp.float32),
    )(acc_p.reshape(NC, ROWS, 128), deg_p.reshape(NC, ROWS, 128),
      d_pad.reshape(ROWS, 128))
    return out[0, 0]


@jax.jit
def kernel(u_out, v_out, u_tgt, v_tgt, edge_index):
    d = (v_out - v_tgt) - (u_out - u_tgt)
    d_pad = jnp.zeros((NP,), jnp.float32).at[:N_NODES].set(d)
    ei = edge_index.astype(jnp.int32)
    zeros = jnp.zeros((NP,), jnp.float32)
    acc_p, deg_p = _edge_accumulate(ei, d_pad, zeros)
    return _masked_loss(acc_p, deg_p, d_pad)


# speedup vs baseline: 2.5060x; 1.0212x over previous
import functools

import jax
import jax.numpy as jnp
from jax import lax
from jax.experimental import pallas as pl
from jax.experimental.pallas import tpu as pltpu
from jax.experimental.pallas import tpu_sc as plsc

N_NODES = 100000
N_EDGES = 6400000
NP = 100352
ROWS = NP // 128
C = 128
K = 16
NC = 2
NS = 16
NW = NC * NS
TOTAL_CHUNKS = N_EDGES // C
TOTAL_GROUPS = TOTAL_CHUNKS // K
BASE_G = TOTAL_GROUPS // NW
EXTRA_G = TOTAL_GROUPS - BASE_G * NW
SLICE = NP // NS


def _edge_accumulate(ei, d_pad, zeros):
    mesh = plsc.VectorSubcoreMesh(core_axis_name="c", subcore_axis_name="s")

    @functools.partial(
        pl.kernel,
        out_type=(
            jax.ShapeDtypeStruct((NC, NP), jnp.float32),
            jax.ShapeDtypeStruct((NC, NP), jnp.float32),
        ),
        mesh=mesh,
        compiler_params=pltpu.CompilerParams(needs_layout_passes=False),
        scratch_types=[
            pltpu.VMEM((NP,), jnp.float32),
            pltpu.VMEM((K * C,), jnp.int32),
            pltpu.VMEM((K * C,), jnp.int32),
            pltpu.VMEM((K * C,), jnp.float32),
            pltpu.VMEM((K * C,), jnp.int32),
            pltpu.VMEM((K * C,), jnp.int32),
            pltpu.VMEM((K * C,), jnp.float32),
            pltpu.VMEM((C,), jnp.float32),
            pltpu.VMEM_SHARED((NP,), jnp.float32),
            pltpu.VMEM_SHARED((NP,), jnp.float32),
            pltpu.SemaphoreType.DMA,
            pltpu.SemaphoreType.DMA,
        ],
    )
    def k(ei_hbm, d_hbm, z_hbm, acc_out, deg_out,
          d_v, src_a, dst_a, vals_a, src_b, dst_b, vals_b,
          ones, acc_sh, deg_sh, ssem, lsem):
        cid = lax.axis_index("c")
        sid = lax.axis_index("s")
        wid = sid * NC + cid

        off = sid * SLICE
        h1 = pltpu.async_copy(z_hbm.at[pl.ds(off, SLICE)],
                              acc_sh.at[pl.ds(off, SLICE)], lsem)
        h2 = pltpu.async_copy(z_hbm.at[pl.ds(off, SLICE)],
                              deg_sh.at[pl.ds(off, SLICE)], lsem)
        h3 = pltpu.async_copy(d_hbm, d_v, lsem)
        for j in range(C // 16):
            ones[pl.ds(j * 16, 16)] = jnp.ones((16,), jnp.float32)
        h1.wait()
        h2.wait()
        h3.wait()
        plsc.subcore_barrier()

        g0 = wid * BASE_G + jnp.minimum(wid, EXTRA_G)
        n_groups = BASE_G + (wid < EXTRA_G).astype(jnp.int32)

        def loadgather(src_r, dst_r, vals_r, g_rel):
            base = (g0 + g_rel) * (K * C)
            h1 = pltpu.async_copy(ei_hbm.at[0, pl.ds(base, K * C)], src_r, lsem)
            h2 = pltpu.async_copy(ei_hbm.at[1, pl.ds(base, K * C)], dst_r, lsem)
            h1.wait()
            h2.wait()
            for t in range(K * C // 16):
                idx = src_r[pl.ds(t * 16, 16)]
                vals_r[pl.ds(t * 16, 16)] = plsc.load_gather(d_v, [idx])

        def fire(dst_r, vals_r):
            for j in range(K):
                pltpu.async_copy(vals_r.at[pl.ds(j * C, C)],
                                 acc_sh.at[dst_r.at[pl.ds(j * C, C)]],
                                 ssem, add=True)
                pltpu.async_copy(ones, deg_sh.at[dst_r.at[pl.ds(j * C, C)]],
                                 ssem, add=True)

        def drain():
            for _ in range(2 * K):
                pltpu.make_async_copy(d_hbm.at[pl.ds(0, C)], ones, ssem).wait()

        n_pairs = n_groups // 2
        tail = n_groups - 2 * n_pairs

        @pl.loop(0, n_pairs)
        def _(gg):
            loadgather(src_a, dst_a, vals_a, 2 * gg)

            @pl.when(gg > 0)
            def _():
                drain()

            fire(dst_a, vals_a)
            loadgather(src_b, dst_b, vals_b, 2 * gg + 1)
            drain()
            fire(dst_b, vals_b)

        drain()

        @pl.when(tail == 1)
        def _():
            loadgather(src_a, dst_a, vals_a, 2 * n_pairs)
            fire(dst_a, vals_a)
            drain()

        plsc.subcore_barrier()
        h1 = pltpu.async_copy(acc_sh.at[pl.ds(off, SLICE)],
                              acc_out.at[cid, pl.ds(off, SLICE)], lsem)
        h2 = pltpu.async_copy(deg_sh.at[pl.ds(off, SLICE)],
                              deg_out.at[cid, pl.ds(off, SLICE)], lsem)
        h1.wait()
        h2.wait()

    return k(ei, d_pad, zeros)


def _loss_body(acc_ref, deg_ref, d_ref, out_ref):
    acc = acc_ref[0] + acc_ref[1]
    deg = deg_ref[0] + deg_ref[1]
    d = d_ref[...]
    w = (deg > 0).astype(jnp.float32)
    vort = acc / jnp.maximum(deg, 1.0) - d
    num = jnp.sum(jnp.abs(vort) * w)
    den = jnp.maximum(jnp.sum(w), 1.0)
    out_ref[...] = jnp.full((1, 1), num / den, jnp.float32)


def _masked_loss(acc_p, deg_p, d_pad):
    out = pl.pallas_call(
        _loss_body,
        out_shape=jax.ShapeDtypeStruct((1, 1), jnp.float32),
    )(acc_p.reshape(NC, ROWS, 128), deg_p.reshape(NC, ROWS, 128),
      d_pad.reshape(ROWS, 128))
    return out[0, 0]


@jax.jit
def kernel(u_out, v_out, u_tgt, v_tgt, edge_index):
    d = (v_out - v_tgt) - (u_out - u_tgt)
    d_pad = jnp.zeros((NP,), jnp.float32).at[:N_NODES].set(d)
    ei = edge_index.astype(jnp.int32)
    zeros = jnp.zeros((NP,), jnp.float32)
    acc_p, deg_p = _edge_accumulate(ei, d_pad, zeros)
    return _masked_loss(acc_p, deg_p, d_pad)
